# baseline (speedup 1.0000x reference)
import jax
import jax.numpy as jnp
from jax.experimental import pallas as pl


def _norm_k(src, dst, ew, n, fill):
    loop = jnp.arange(n, dtype=src.dtype)
    s = jnp.concatenate([src, loop])
    d = jnp.concatenate([dst, loop])
    w = jnp.concatenate([ew, jnp.full((n,), fill, ew.dtype)])
    deg = jnp.zeros((n,), ew.dtype).at[d].add(w)
    dinv = jnp.where(deg > 0, 1.0 / jnp.sqrt(deg), 0.0)
    return s, d, dinv[s] * w * dinv[d]


def _prop_k(x, s, d, nw, n):
    msg = x[s] * nw[:, None]
    return jnp.zeros((n, x.shape[1]), x.dtype).at[d].add(msg)


def _gcn_k(x, s, d, nw, n, W, b):
    return _prop_k(x @ W, s, d, nw, n) + b


def _copy_kernel(x_ref, o_ref):
    o_ref[...] = x_ref[...]


def kernel(x, y, edge_idx, edge_wt, W1, W2, Wmu, bmu, Wvar, bvar,
           Wg1, bg1, Wr1, Wg2, bg2, Wr2, Wg3, bg3, Wr3):
    n = x.shape[0]
    src, dst = edge_idx[0], edge_idx[1]
    s, d, nw = _norm_k(src, dst, edge_wt, n, 1.0)
    si, di, nwi = _norm_k(src, dst, edge_wt, n, 2.0)
    theta = jnp.concatenate([x, y], axis=-1)
    h = jax.nn.relu(theta @ W1)
    for _ in range(3):
        h = _prop_k(h, s, d, nw, n)
    theta = jax.nn.relu(h @ W2)
    mu = _gcn_k(theta, si, di, nwi, n, Wmu, bmu)
    logvar = _gcn_k(theta, si, di, nwi, n, Wvar, bvar)
    noise = jax.random.uniform(jax.random.key(1), mu.shape, mu.dtype)
    z = mu + noise * jnp.exp(0.5 * logvar)
    h1 = jax.nn.relu(_gcn_k(z, s, d, nw, n, Wg1, bg1) + z @ Wr1)
    h2 = jax.nn.relu(_gcn_k(h1, s, d, nw, n, Wg2, bg2) + h1 @ Wr2)
    out = jax.nn.relu(_gcn_k(h2, s, d, nw, n, Wg3, bg3) + h2 @ Wr3)
    out = pl.pallas_call(
        _copy_kernel,
        grid=(100,),
        in_specs=[pl.BlockSpec((1000, 1), lambda i: (i, 0))],
        out_specs=pl.BlockSpec((1000, 1), lambda i: (i, 0)),
        out_shape=jax.ShapeDtypeStruct(out.shape, out.dtype))(out)
    return out


# speedup vs baseline: 20.3371x; 20.3371x over previous
import functools

import jax
import jax.numpy as jnp
from jax import lax
from jax.experimental import pallas as pl
from jax.experimental.pallas import tpu as pltpu
from jax.experimental.pallas import tpu_sc as plsc

_NC = 2
_NS = 16
_NW = _NC * _NS
_F = 16
_SB = 8
_CHUNK = _SB * 128
_ZR = 128

_f32 = jnp.float32
_i32 = jnp.int32


def _mesh():
    return plsc.VectorSubcoreMesh(core_axis_name="c", subcore_axis_name="s")


def _zero_acc(zer_v, acc_sh, sid, nt):
    def _zf(r, c):
        zer_v[r] = jnp.zeros((_F,), _f32)
        return c
    lax.fori_loop(0, _ZR, _zf, 0)
    for k in range(nt // _ZR):
        pltpu.sync_copy(zer_v, acc_sh.at[pl.ds(sid * nt + k * _ZR, _ZR)])


def _scale_rows(rows_v, w_v):
    def _mul(gi, c):
        base = gi * 16
        for j in range(16):
            wb = plsc.load_gather(w_v, [jnp.full((16,), base + j, _i32)])
            rows_v[base + j] = rows_v[base + j] * wb
        return c
    lax.fori_loop(0, _CHUNK // 16, _mul, 0)


def _copy_out(acc_sh, out0, out1, cid, sid, nt):
    @pl.when(cid == 0)
    def _():
        pltpu.sync_copy(acc_sh.at[pl.ds(sid * nt, nt)],
                        out0.at[pl.ds(sid * nt, nt)])

    @pl.when(cid == 1)
    def _():
        pltpu.sync_copy(acc_sh.at[pl.ds(sid * nt, nt)],
                        out1.at[pl.ds(sid * nt, nt)])


def _make_prop(n, e_rows):
    nt = n // _NS
    rpt = e_rows // _NW
    n_iters = rpt // _SB

    def body(xp, s2, d2, w1, out0, out1, s_v, d_v, w_v, rows_v, zer_v,
             acc_sh, gsem):
        cid = lax.axis_index("c")
        sid = lax.axis_index("s")
        _zero_acc(zer_v, acc_sh, sid, nt)
        plsc.subcore_barrier()
        base = (cid * _NS + sid) * rpt

        def _eloop(i, c):
            roff = base + i * _SB
            pltpu.sync_copy(s2.at[pl.ds(roff, _SB)], s_v)
            pltpu.sync_copy(d2.at[pl.ds(roff, _SB)], d_v)
            pltpu.sync_copy(w1.at[pl.ds(roff * 128, _CHUNK)], w_v)
            cps = [pltpu.async_copy(xp.at[s_v.at[g]],
                                    rows_v.at[pl.ds(g * 128, 128)], gsem)
                   for g in range(_SB)]
            for cp in cps:
                cp.wait()
            _scale_rows(rows_v, w_v)
            for g in range(_SB):
                pltpu.sync_copy(rows_v.at[pl.ds(g * 128, 128)],
                                acc_sh.at[d_v.at[g]], add=True)
            return c

        lax.fori_loop(0, n_iters, _eloop, 0)
        plsc.subcore_barrier()
        _copy_out(acc_sh, out0, out1, cid, sid, nt)

    return pl.kernel(
        body,
        out_type=(jax.ShapeDtypeStruct((n, _F), _f32),
                  jax.ShapeDtypeStruct((n, _F), _f32)),
        mesh=_mesh(),
        compiler_params=pltpu.CompilerParams(needs_layout_passes=False, use_tc_tiling_on_sc=False),
        scratch_types=[
            pltpu.VMEM((_SB, 128), _i32),
            pltpu.VMEM((_SB, 128), _i32),
            pltpu.VMEM((_CHUNK,), _f32),
            pltpu.VMEM((_CHUNK, _F), _f32),
            pltpu.VMEM((_ZR, _F), _f32),
            pltpu.VMEM_SHARED((n, _F), _f32),
            pltpu.SemaphoreType.DMA,
        ],
    )


def _make_deg(n, e_rows):
    nt = n // _NS
    rpt = e_rows // _NW
    n_iters = rpt // _SB

    def body(d2, w1, out0, out1, d_v, w_v, rows_v, zer_v, acc_sh):
        cid = lax.axis_index("c")
        sid = lax.axis_index("s")
        _zero_acc(zer_v, acc_sh, sid, nt)
        plsc.subcore_barrier()
        base = (cid * _NS + sid) * rpt

        def _eloop(i, c):
            roff = base + i * _SB
            pltpu.sync_copy(d2.at[pl.ds(roff, _SB)], d_v)
            pltpu.sync_copy(w1.at[pl.ds(roff * 128, _CHUNK)], w_v)

            def _st(gi, c2):
                gb = gi * 16
                for j in range(16):
                    wb = plsc.load_gather(w_v, [jnp.full((16,), gb + j, _i32)])
                    rows_v[gb + j] = wb
                return c2

            lax.fori_loop(0, _CHUNK // 16, _st, 0)
            for g in range(_SB):
                pltpu.sync_copy(rows_v.at[pl.ds(g * 128, 128)],
                                acc_sh.at[d_v.at[g]], add=True)
            return c

        lax.fori_loop(0, n_iters, _eloop, 0)
        plsc.subcore_barrier()
        _copy_out(acc_sh, out0, out1, cid, sid, nt)

    return pl.kernel(
        body,
        out_type=(jax.ShapeDtypeStruct((n, _F), _f32),
                  jax.ShapeDtypeStruct((n, _F), _f32)),
        mesh=_mesh(),
        compiler_params=pltpu.CompilerParams(needs_layout_passes=False, use_tc_tiling_on_sc=False),
        scratch_types=[
            pltpu.VMEM((_SB, 128), _i32),
            pltpu.VMEM((_CHUNK,), _f32),
            pltpu.VMEM((_CHUNK, _F), _f32),
            pltpu.VMEM((_ZR, _F), _f32),
            pltpu.VMEM_SHARED((n, _F), _f32),
        ],
    )


_B = 2048


def _tc_call(body, n, ins, out_feats):
    grid = (n // _B,)

    def spec(a):
        if a.shape[0] == n:
            blk = (_B,) + a.shape[1:]
            return pl.BlockSpec(blk, lambda i: (i,) + (0,) * (a.ndim - 1))
        return pl.BlockSpec(a.shape, lambda i: (0,) * a.ndim)

    out_shape = tuple(jax.ShapeDtypeStruct((n, f), _f32) for f in out_feats)
    out_specs = tuple(pl.BlockSpec((_B, f), lambda i: (i, 0))
                      for f in out_feats)
    return pl.pallas_call(
        body, grid=grid,
        in_specs=[spec(a) for a in ins],
        out_specs=out_specs,
        out_shape=out_shape)(*ins)


def _t1_body(x, y, p0, p1, w1, dinv1, dinv2, h0, g0):
    deg = p0[:, :1] + p1[:, :1]
    d1 = lax.rsqrt(deg + 1.0)
    d2 = lax.rsqrt(deg + 2.0)
    h = jnp.maximum(x[...] * w1[0:1, :] + y[...] * w1[1:2, :], 0.0)
    dinv1[...] = d1
    dinv2[...] = d2
    h0[...] = h
    g0[...] = d1 * h


def _t2_body(p0, p1, hp, d1r, h_o, g_o):
    d1 = d1r[...]
    h = d1 * (p0[...] + p1[...]) + (d1 * d1) * hp[...]
    h_o[...] = h
    g_o[...] = d1 * h


def _t3_body(p0, p1, hp, d1r, d2r, w2, wmu, wvar, g_o, tmu_o, tvar_o):
    d1 = d1r[...]
    d2 = d2r[...]
    h3 = d1 * (p0[...] + p1[...]) + (d1 * d1) * hp[...]
    th = jnp.maximum(jnp.dot(h3, w2[...], preferred_element_type=_f32), 0.0)
    tmu = jnp.dot(th, wmu[...], preferred_element_type=_f32)
    tvar = jnp.dot(th, wvar[...], preferred_element_type=_f32)
    g_o[...] = jnp.concatenate(
        [d2 * tmu, d2 * tvar, jnp.zeros((tmu.shape[0], _F - 2), _f32)], axis=1)
    tmu_o[...] = tmu
    tvar_o[...] = tvar


def _t4_body(pm0, pm1, tmu, tvar, d2r, d1r, noise, bmu, bvar, wg1, z_o, g1_o):
    d2 = d2r[...]
    d1 = d1r[...]
    mu = d2 * (pm0[:, :1] + pm1[:, :1]) + 2.0 * (d2 * d2) * tmu[...] + bmu[...]
    lv = d2 * (pm0[:, 1:2] + pm1[:, 1:2]) + 2.0 * (d2 * d2) * tvar[...] + bvar[...]
    z = mu + noise[...] * jnp.exp(0.5 * lv)
    z_o[...] = z
    g1_o[...] = d1 * (z * wg1[0:1, :])


def _t5_body(q0, q1, z, d1r, wg1, bg1, wr1, wg2, h1_o, g2a_o, g2b_o):
    d1 = d1r[...]
    zv = z[...]
    h1 = jnp.maximum(d1 * (q0[...] + q1[...]) + (d1 * d1) * (zv * wg1[0:1, :])
                     + bg1[...] + zv * wr1[0:1, :], 0.0)
    t2 = jnp.dot(h1, wg2[...], preferred_element_type=_f32)
    h1_o[...] = h1
    g2a_o[...] = d1 * t2[:, :_F]
    g2b_o[...] = d1 * t2[:, _F:]


def _t6_body(r0a, r1a, r0b, r1b, h1, d1r, wg2, bg2, wr2, wg3,
             h2_o, t3_o, g3_o):
    d1 = d1r[...]
    p2 = jnp.concatenate([r0a[...] + r1a[...], r0b[...] + r1b[...]], axis=1)
    t2 = jnp.dot(h1[...], wg2[...], preferred_element_type=_f32)
    h2 = jnp.maximum(d1 * p2 + (d1 * d1) * t2 + bg2[...]
                     + jnp.dot(h1[...], wr2[...], preferred_element_type=_f32),
                     0.0)
    t3 = jnp.dot(h2, wg3[...], preferred_element_type=_f32)
    h2_o[...] = h2
    t3_o[...] = t3
    g3_o[...] = jnp.concatenate(
        [d1 * t3, jnp.zeros((t3.shape[0], _F - 1), _f32)], axis=1)


def _t7_body(u0, u1, h2, t3, d1r, bg3, wr3, out_o):
    d1 = d1r[...]
    out_o[...] = jnp.maximum(
        d1 * (u0[:, :1] + u1[:, :1]) + (d1 * d1) * t3[...] + bg3[...]
        + jnp.dot(h2[...], wr3[...], preferred_element_type=_f32), 0.0)


def kernel(x, y, edge_idx, edge_wt, W1, W2, Wmu, bmu, Wvar, bvar,
           Wg1, bg1, Wr1, Wg2, bg2, Wr2, Wg3, bg3, Wr3):
    n = x.shape[0]
    npad = -(-n // _B) * _B
    src, dst = edge_idx[0], edge_idx[1]
    e = src.shape[0]

    def _padn(a):
        return jnp.concatenate(
            [a, jnp.zeros((npad - n,) + a.shape[1:], a.dtype)])

    per_tile = -(-e // (_NW * _CHUNK)) * _CHUNK
    e_pad = _NW * per_tile
    padz = e_pad - e
    sp = jnp.concatenate([src, jnp.zeros((padz,), _i32)]).reshape(-1, 128)
    dp = jnp.concatenate([dst, jnp.zeros((padz,), _i32)]).reshape(-1, 128)
    wp = jnp.concatenate([edge_wt, jnp.zeros((padz,), _f32)])
    e_rows = e_pad // 128

    prop = _make_prop(npad, e_rows)
    deg = _make_deg(npad, e_rows)

    noise = _padn(jax.random.uniform(jax.random.key(1), (n, 1), _f32))
    xq, yq = _padn(x), _padn(y)
    bmu2 = bmu.reshape(1, 1)
    bvar2 = bvar.reshape(1, 1)
    bg1_2 = bg1.reshape(1, _F)
    bg2_2 = bg2.reshape(1, 2 * _F)
    bg3_2 = bg3.reshape(1, 1)

    dg0, dg1 = deg(dp, wp)
    dinv1, dinv2, h0, g0 = _tc_call(
        _t1_body, npad, [xq, yq, dg0, dg1, W1], (1, 1, _F, _F))

    h, g = h0, g0
    for _ in range(2):
        p0, p1 = prop(g, sp, dp, wp)
        h, g = _tc_call(_t2_body, npad, [p0, p1, h, dinv1], (_F, _F))
    p0, p1 = prop(g, sp, dp, wp)
    gmv, tmu, tvar = _tc_call(
        _t3_body, npad, [p0, p1, h, dinv1, dinv2, W2, Wmu, Wvar], (_F, 1, 1))

    pm0, pm1 = prop(gmv, sp, dp, wp)
    z, g1 = _tc_call(
        _t4_body, npad,
        [pm0, pm1, tmu, tvar, dinv2, dinv1, noise, bmu2, bvar2, Wg1], (1, _F))

    q0, q1 = prop(g1, sp, dp, wp)
    h1, g2a, g2b = _tc_call(
        _t5_body, npad, [q0, q1, z, dinv1, Wg1, bg1_2, Wr1, Wg2], (_F, _F, _F))

    r0a, r1a = prop(g2a, sp, dp, wp)
    r0b, r1b = prop(g2b, sp, dp, wp)
    h2, t3, g3 = _tc_call(
        _t6_body, npad, [r0a, r1a, r0b, r1b, h1, dinv1, Wg2, bg2_2, Wr2, Wg3],
        (2 * _F, 1, _F))

    u0, u1 = prop(g3, sp, dp, wp)
    (out,) = _tc_call(
        _t7_body, npad, [u0, u1, h2, t3, dinv1, bg3_2, Wr3], (1,))
    return out[:n]


# speedup vs baseline: 30.1421x; 1.4821x over previous
import functools

import jax
import jax.numpy as jnp
from jax import lax
from jax.experimental import pallas as pl
from jax.experimental.pallas import tpu as pltpu
from jax.experimental.pallas import tpu_sc as plsc

_NC = 2
_NS = 16
_NW = _NC * _NS
_F = 16
_SB = 8
_CHUNK = _SB * 128
_ZR = 128

_f32 = jnp.float32
_i32 = jnp.int32


def _mesh():
    return plsc.VectorSubcoreMesh(core_axis_name="c", subcore_axis_name="s")


def _zero_acc(zer_v, acc_sh, sid, nt):
    def _zf(r, c):
        zer_v[r] = jnp.zeros((_F,), _f32)
        return c
    lax.fori_loop(0, _ZR, _zf, 0)
    for k in range(nt // _ZR):
        pltpu.sync_copy(zer_v, acc_sh.at[pl.ds(sid * nt + k * _ZR, _ZR)])


def _scale_rows(rows_v, w_v):
    def _mul(gi, c):
        base = gi * 16
        for j in range(16):
            wb = plsc.load_gather(w_v, [jnp.full((16,), base + j, _i32)])
            rows_v[base + j] = rows_v[base + j] * wb
        return c
    lax.fori_loop(0, _CHUNK // 16, _mul, 0)


def _copy_out(acc_sh, out0, out1, cid, sid, nt):
    @pl.when(cid == 0)
    def _():
        pltpu.sync_copy(acc_sh.at[pl.ds(sid * nt, nt)],
                        out0.at[pl.ds(sid * nt, nt)])

    @pl.when(cid == 1)
    def _():
        pltpu.sync_copy(acc_sh.at[pl.ds(sid * nt, nt)],
                        out1.at[pl.ds(sid * nt, nt)])


_SBH = 4
_CH = _SBH * 128
_DNUMS = lax.GatherDimensionNumbers(
    offset_dims=(), collapsed_slice_dims=(0,), start_index_map=(0,))


def _bcast(vec, j):
    idx = jnp.full((16, 1), j, _i32)
    return lax.gather(vec, idx, _DNUMS, (1,),
                      mode=lax.GatherScatterMode.PROMISE_IN_BOUNDS)


def _zero_acc_async(zer_v, acc_sh, sid, nt, sem):
    def _zf(r, c):
        zer_v[r] = jnp.zeros((_F,), _f32)
        return c
    lax.fori_loop(0, _ZR, _zf, 0)
    nz = nt // _ZR

    def _fire(k, c):
        pltpu.async_copy(zer_v, acc_sh.at[pl.ds(sid * nt + k * _ZR, _ZR)], sem)
        return c
    lax.fori_loop(0, nz, _fire, 0)

    def _drain(k, c):
        pltpu.make_async_copy(
            zer_v, acc_sh.at[pl.ds(sid * nt + k * _ZR, _ZR)], sem).wait()
        return c
    lax.fori_loop(0, nz, _drain, 0)


def _make_prop(n, e_rows):
    nt = n // _NS
    rpt = e_rows // _NW
    npair = rpt // (2 * _SBH)

    def body(xp, s2, d2, w1, out0, out1,
             sA, sB, dA, dB, wA, wB, rowsA, rowsB, zer_v, acc_sh,
             gsemA, gsemB, ssemA, ssemB):
        cid = lax.axis_index("c")
        sid = lax.axis_index("s")
        _zero_acc_async(zer_v, acc_sh, sid, nt, gsemA)
        plsc.subcore_barrier()
        base = (cid * _NS + sid) * rpt

        def load_idx(k, s_v, d_v, w_v):
            roff = base + k * _SBH
            pltpu.sync_copy(s2.at[pl.ds(roff, _SBH)], s_v)
            pltpu.sync_copy(d2.at[pl.ds(roff, _SBH)], d_v)
            pltpu.sync_copy(w1.at[pl.ds(roff * 128, _CH)], w_v)

        def fire_gathers(s_v, rows_v, sem):
            for g in range(_SBH):
                pltpu.async_copy(xp.at[s_v.at[g]],
                                 rows_v.at[pl.ds(g * 128, 128)], sem)

        def drain_gathers(s_v, rows_v, sem):
            for g in range(_SBH):
                pltpu.make_async_copy(xp.at[s_v.at[g]],
                                      rows_v.at[pl.ds(g * 128, 128)],
                                      sem).wait()

        def fire_scatters(rows_v, d_v, sem):
            for g in range(_SBH):
                pltpu.async_copy(rows_v.at[pl.ds(g * 128, 128)],
                                 acc_sh.at[d_v.at[g]], sem, add=True)

        def drain_scatters(rows_v, d_v, sem):
            for g in range(_SBH):
                pltpu.make_async_copy(rows_v.at[pl.ds(g * 128, 128)],
                                      acc_sh.at[d_v.at[g]], sem).wait()

        def compute(rows_v, w_v):
            def _grp(gi, c):
                gb = gi * 16
                w16 = w_v[pl.ds(gb, 16)]
                for j in range(16):
                    rows_v[gb + j] = rows_v[gb + j] * _bcast(w16, j)
                return c
            lax.fori_loop(0, _CH // 16, _grp, 0)

        load_idx(0, sA, dA, wA)
        fire_gathers(sA, rowsA, gsemA)

        def _pair(i, c):
            k1 = 2 * i + 1
            k2 = 2 * i + 2
            @pl.when(i > 0)
            def _():
                drain_scatters(rowsB, dB, ssemB)
            load_idx(k1, sB, dB, wB)
            fire_gathers(sB, rowsB, gsemB)
            drain_gathers(sA, rowsA, gsemA)
            compute(rowsA, wA)
            fire_scatters(rowsA, dA, ssemA)
            @pl.when(k2 < 2 * npair)
            def _():
                drain_scatters(rowsA, dA, ssemA)
                load_idx(k2, sA, dA, wA)
                fire_gathers(sA, rowsA, gsemA)
            drain_gathers(sB, rowsB, gsemB)
            compute(rowsB, wB)
            fire_scatters(rowsB, dB, ssemB)
            return c

        lax.fori_loop(0, npair, _pair, 0)
        drain_scatters(rowsA, dA, ssemA)
        drain_scatters(rowsB, dB, ssemB)
        plsc.subcore_barrier()
        _copy_out(acc_sh, out0, out1, cid, sid, nt)

    return pl.kernel(
        body,
        out_type=(jax.ShapeDtypeStruct((n, _F), _f32),
                  jax.ShapeDtypeStruct((n, _F), _f32)),
        mesh=_mesh(),
        compiler_params=pltpu.CompilerParams(needs_layout_passes=False, use_tc_tiling_on_sc=False),
        scratch_types=[
            pltpu.VMEM((_SBH, 128), _i32),
            pltpu.VMEM((_SBH, 128), _i32),
            pltpu.VMEM((_SBH, 128), _i32),
            pltpu.VMEM((_SBH, 128), _i32),
            pltpu.VMEM((_CH,), _f32),
            pltpu.VMEM((_CH,), _f32),
            pltpu.VMEM((_CH, _F), _f32),
            pltpu.VMEM((_CH, _F), _f32),
            pltpu.VMEM((_ZR, _F), _f32),
            pltpu.VMEM_SHARED((n, _F), _f32),
            pltpu.SemaphoreType.DMA,
            pltpu.SemaphoreType.DMA,
            pltpu.SemaphoreType.DMA,
            pltpu.SemaphoreType.DMA,
        ],
    )


def _make_deg(n, e_rows):
    nt = n // _NS
    rpt = e_rows // _NW
    n_iters = rpt // _SB

    def body(d2, w1, out0, out1, d_v, w_v, rows_v, zer_v, acc_sh):
        cid = lax.axis_index("c")
        sid = lax.axis_index("s")
        _zero_acc(zer_v, acc_sh, sid, nt)
        plsc.subcore_barrier()
        base = (cid * _NS + sid) * rpt

        def _eloop(i, c):
            roff = base + i * _SB
            pltpu.sync_copy(d2.at[pl.ds(roff, _SB)], d_v)
            pltpu.sync_copy(w1.at[pl.ds(roff * 128, _CHUNK)], w_v)

            def _st(gi, c2):
                gb = gi * 16
                for j in range(16):
                    wb = plsc.load_gather(w_v, [jnp.full((16,), gb + j, _i32)])
                    rows_v[gb + j] = wb
                return c2

            lax.fori_loop(0, _CHUNK // 16, _st, 0)
            for g in range(_SB):
                pltpu.sync_copy(rows_v.at[pl.ds(g * 128, 128)],
                                acc_sh.at[d_v.at[g]], add=True)
            return c

        lax.fori_loop(0, n_iters, _eloop, 0)
        plsc.subcore_barrier()
        _copy_out(acc_sh, out0, out1, cid, sid, nt)

    return pl.kernel(
        body,
        out_type=(jax.ShapeDtypeStruct((n, _F), _f32),
                  jax.ShapeDtypeStruct((n, _F), _f32)),
        mesh=_mesh(),
        compiler_params=pltpu.CompilerParams(needs_layout_passes=False, use_tc_tiling_on_sc=False),
        scratch_types=[
            pltpu.VMEM((_SB, 128), _i32),
            pltpu.VMEM((_CHUNK,), _f32),
            pltpu.VMEM((_CHUNK, _F), _f32),
            pltpu.VMEM((_ZR, _F), _f32),
            pltpu.VMEM_SHARED((n, _F), _f32),
        ],
    )


_B = 2048


def _tc_call(body, n, ins, out_feats):
    grid = (n // _B,)

    def spec(a):
        if a.shape[0] == n:
            blk = (_B,) + a.shape[1:]
            return pl.BlockSpec(blk, lambda i: (i,) + (0,) * (a.ndim - 1))
        return pl.BlockSpec(a.shape, lambda i: (0,) * a.ndim)

    out_shape = tuple(jax.ShapeDtypeStruct((n, f), _f32) for f in out_feats)
    out_specs = tuple(pl.BlockSpec((_B, f), lambda i: (i, 0))
                      for f in out_feats)
    return pl.pallas_call(
        body, grid=grid,
        in_specs=[spec(a) for a in ins],
        out_specs=out_specs,
        out_shape=out_shape)(*ins)


def _t1_body(x, y, p0, p1, w1, dinv1, dinv2, h0, g0):
    deg = p0[:, :1] + p1[:, :1]
    d1 = lax.rsqrt(deg + 1.0)
    d2 = lax.rsqrt(deg + 2.0)
    h = jnp.maximum(x[...] * w1[0:1, :] + y[...] * w1[1:2, :], 0.0)
    dinv1[...] = d1
    dinv2[...] = d2
    h0[...] = h
    g0[...] = d1 * h


def _t2_body(p0, p1, hp, d1r, h_o, g_o):
    d1 = d1r[...]
    h = d1 * (p0[...] + p1[...]) + (d1 * d1) * hp[...]
    h_o[...] = h
    g_o[...] = d1 * h


def _t3_body(p0, p1, hp, d1r, d2r, w2, wmu, wvar, g_o, tmu_o, tvar_o):
    d1 = d1r[...]
    d2 = d2r[...]
    h3 = d1 * (p0[...] + p1[...]) + (d1 * d1) * hp[...]
    th = jnp.maximum(jnp.dot(h3, w2[...], preferred_element_type=_f32), 0.0)
    tmu = jnp.dot(th, wmu[...], preferred_element_type=_f32)
    tvar = jnp.dot(th, wvar[...], preferred_element_type=_f32)
    g_o[...] = jnp.concatenate(
        [d2 * tmu, d2 * tvar, jnp.zeros((tmu.shape[0], _F - 2), _f32)], axis=1)
    tmu_o[...] = tmu
    tvar_o[...] = tvar


def _t4_body(pm0, pm1, tmu, tvar, d2r, d1r, noise, bmu, bvar, wg1, z_o, g1_o):
    d2 = d2r[...]
    d1 = d1r[...]
    mu = d2 * (pm0[:, :1] + pm1[:, :1]) + 2.0 * (d2 * d2) * tmu[...] + bmu[...]
    lv = d2 * (pm0[:, 1:2] + pm1[:, 1:2]) + 2.0 * (d2 * d2) * tvar[...] + bvar[...]
    z = mu + noise[...] * jnp.exp(0.5 * lv)
    z_o[...] = z
    g1_o[...] = d1 * (z * wg1[0:1, :])


def _t5_body(q0, q1, z, d1r, wg1, bg1, wr1, wg2, h1_o, g2a_o, g2b_o):
    d1 = d1r[...]
    zv = z[...]
    h1 = jnp.maximum(d1 * (q0[...] + q1[...]) + (d1 * d1) * (zv * wg1[0:1, :])
                     + bg1[...] + zv * wr1[0:1, :], 0.0)
    t2 = jnp.dot(h1, wg2[...], preferred_element_type=_f32)
    h1_o[...] = h1
    g2a_o[...] = d1 * t2[:, :_F]
    g2b_o[...] = d1 * t2[:, _F:]


def _t6_body(r0a, r1a, r0b, r1b, h1, d1r, wg2, bg2, wr2, wg3,
             h2_o, t3_o, g3_o):
    d1 = d1r[...]
    p2 = jnp.concatenate([r0a[...] + r1a[...], r0b[...] + r1b[...]], axis=1)
    t2 = jnp.dot(h1[...], wg2[...], preferred_element_type=_f32)
    h2 = jnp.maximum(d1 * p2 + (d1 * d1) * t2 + bg2[...]
                     + jnp.dot(h1[...], wr2[...], preferred_element_type=_f32),
                     0.0)
    t3 = jnp.dot(h2, wg3[...], preferred_element_type=_f32)
    h2_o[...] = h2
    t3_o[...] = t3
    g3_o[...] = jnp.concatenate(
        [d1 * t3, jnp.zeros((t3.shape[0], _F - 1), _f32)], axis=1)


def _t7_body(u0, u1, h2, t3, d1r, bg3, wr3, out_o):
    d1 = d1r[...]
    out_o[...] = jnp.maximum(
        d1 * (u0[:, :1] + u1[:, :1]) + (d1 * d1) * t3[...] + bg3[...]
        + jnp.dot(h2[...], wr3[...], preferred_element_type=_f32), 0.0)


def kernel(x, y, edge_idx, edge_wt, W1, W2, Wmu, bmu, Wvar, bvar,
           Wg1, bg1, Wr1, Wg2, bg2, Wr2, Wg3, bg3, Wr3):
    n = x.shape[0]
    npad = -(-n // _B) * _B
    src, dst = edge_idx[0], edge_idx[1]
    e = src.shape[0]

    def _padn(a):
        return jnp.concatenate(
            [a, jnp.zeros((npad - n,) + a.shape[1:], a.dtype)])

    per_tile = -(-e // (_NW * _CHUNK)) * _CHUNK
    e_pad = _NW * per_tile
    padz = e_pad - e
    sp = jnp.concatenate([src, jnp.zeros((padz,), _i32)]).reshape(-1, 128)
    dp = jnp.concatenate([dst, jnp.zeros((padz,), _i32)]).reshape(-1, 128)
    wp = jnp.concatenate([edge_wt, jnp.zeros((padz,), _f32)])
    e_rows = e_pad // 128

    prop = _make_prop(npad, e_rows)
    deg = _make_deg(npad, e_rows)

    noise = _padn(jax.random.uniform(jax.random.key(1), (n, 1), _f32))
    xq, yq = _padn(x), _padn(y)
    bmu2 = bmu.reshape(1, 1)
    bvar2 = bvar.reshape(1, 1)
    bg1_2 = bg1.reshape(1, _F)
    bg2_2 = bg2.reshape(1, 2 * _F)
    bg3_2 = bg3.reshape(1, 1)

    dg0, dg1 = deg(dp, wp)
    dinv1, dinv2, h0, g0 = _tc_call(
        _t1_body, npad, [xq, yq, dg0, dg1, W1], (1, 1, _F, _F))

    h, g = h0, g0
    for _ in range(2):
        p0, p1 = prop(g, sp, dp, wp)
        h, g = _tc_call(_t2_body, npad, [p0, p1, h, dinv1], (_F, _F))
    p0, p1 = prop(g, sp, dp, wp)
    gmv, tmu, tvar = _tc_call(
        _t3_body, npad, [p0, p1, h, dinv1, dinv2, W2, Wmu, Wvar], (_F, 1, 1))

    pm0, pm1 = prop(gmv, sp, dp, wp)
    z, g1 = _tc_call(
        _t4_body, npad,
        [pm0, pm1, tmu, tvar, dinv2, dinv1, noise, bmu2, bvar2, Wg1], (1, _F))

    q0, q1 = prop(g1, sp, dp, wp)
    h1, g2a, g2b = _tc_call(
        _t5_body, npad, [q0, q1, z, dinv1, Wg1, bg1_2, Wr1, Wg2], (_F, _F, _F))

    r0a, r1a = prop(g2a, sp, dp, wp)
    r0b, r1b = prop(g2b, sp, dp, wp)
    h2, t3, g3 = _tc_call(
        _t6_body, npad, [r0a, r1a, r0b, r1b, h1, dinv1, Wg2, bg2_2, Wr2, Wg3],
        (2 * _F, 1, _F))

    u0, u1 = prop(g3, sp, dp, wp)
    (out,) = _tc_call(
        _t7_body, npad, [u0, u1, h2, t3, dinv1, bg3_2, Wr3], (1,))
    return out[:n]
